# baseline (device time: 30319 ns/iter reference)
import jax
import jax.numpy as jnp
from jax import lax
from jax.experimental import pallas as pl
from jax.experimental.pallas import tpu as pltpu

N_DEV = 32
EPS = 1e-5
N_TOT = 4096 * 128
NCH = 8


def kernel(x, Wp):
    xt = jnp.swapaxes(x, 2, 3)
    b, h, c, w = xt.shape
    c2 = Wp.shape[1]
    hc = h // NCH

    def body(
        x_hbm,
        wp_ref,
        out_hbm,
        x_vmem,
        out_stage,
        gather_buf,
        in_sems,
        out_sems,
        send_sems,
        recv_sems,
    ):
        my_pos = lax.axis_index("i")

        barrier_sem = pltpu.get_barrier_semaphore()
        for k in range(1, N_DEV):
            peer = lax.rem(my_pos + k, N_DEV)
            pl.semaphore_signal(
                barrier_sem, inc=1,
                device_id=(peer,), device_id_type=pl.DeviceIdType.MESH,
            )

        in_copies = []
        for i in range(NCH):
            cp = pltpu.make_async_copy(
                x_hbm.at[:, pl.ds(i * hc, hc)],
                x_vmem.at[:, pl.ds(i * hc, hc)],
                in_sems.at[i],
            )
            cp.start()
            in_copies.append(cp)

        s = jnp.zeros((b, c), jnp.float32)
        sq = jnp.zeros((b, c), jnp.float32)
        for i in range(NCH):
            in_copies[i].wait()
            xc = x_vmem[:, i * hc : (i + 1) * hc]
            s = s + jnp.sum(xc, axis=(1, 3))
            sq = sq + jnp.sum(xc * xc, axis=(1, 3))
        gather_buf[0] = jnp.concatenate([s, sq], axis=0)

        pl.semaphore_wait(barrier_sem, N_DEV - 1)

        rdmas = []
        for k in range(1, N_DEV):
            peer = lax.rem(my_pos + k, N_DEV)
            rdma = pltpu.make_async_remote_copy(
                src_ref=gather_buf.at[0],
                dst_ref=gather_buf.at[k],
                send_sem=send_sems.at[k],
                recv_sem=recv_sems.at[k],
                device_id=(peer,),
                device_id_type=pl.DeviceIdType.MESH,
            )
            rdma.start()
            rdmas.append(rdma)
        for rdma in rdmas:
            rdma.wait_send()
        for rdma in rdmas:
            rdma.wait_recv()

        tot = jnp.sum(gather_buf[...], axis=0)
        mean = tot[0:b] / N_TOT
        var = tot[b : 2 * b] / N_TOT - mean * mean
        rstd = lax.rsqrt(var + EPS)
        wp = wp_ref[...].astype(jnp.bfloat16)

        out_copies = [None, None]
        for i in range(NCH):
            slot = i % 2
            if out_copies[slot] is not None:
                out_copies[slot].wait()
            xc = x_vmem[:, i * hc : (i + 1) * hc]
            hn = (
                (xc - mean[:, None, :, None]) * rstd[:, None, :, None]
            ).astype(jnp.bfloat16)
            a = hn * jax.nn.sigmoid(hn)
            a3 = a.reshape(b * hc, c, w)
            out = lax.dot_general(
                a3, wp,
                dimension_numbers=(((1,), (0,)), ((), ())),
                preferred_element_type=jnp.float32,
            )
            out_stage[slot] = out.reshape(b, hc, w, c2).astype(jnp.bfloat16)
            cp = pltpu.make_async_copy(
                out_stage.at[slot],
                out_hbm.at[:, pl.ds(i * hc, hc)],
                out_sems.at[slot],
            )
            cp.start()
            out_copies[slot] = cp
        for cp in out_copies:
            cp.wait()

    return pl.pallas_call(
        body,
        out_shape=jax.ShapeDtypeStruct((b, h, w, c2), jnp.bfloat16),
        in_specs=[
            pl.BlockSpec(memory_space=pl.ANY),
            pl.BlockSpec(memory_space=pltpu.VMEM),
        ],
        out_specs=pl.BlockSpec(memory_space=pl.ANY),
        scratch_shapes=[
            pltpu.VMEM((b, h, c, w), jnp.float32),
            pltpu.VMEM((2, b, hc, w, c2), jnp.bfloat16),
            pltpu.VMEM((N_DEV, 2 * b, c), jnp.float32),
            pltpu.SemaphoreType.DMA((NCH,)),
            pltpu.SemaphoreType.DMA((2,)),
            pltpu.SemaphoreType.DMA((N_DEV,)),
            pltpu.SemaphoreType.DMA((N_DEV,)),
        ],
        compiler_params=pltpu.CompilerParams(
            collective_id=0, vmem_limit_bytes=64 * 1024 * 1024
        ),
    )(xt, Wp)


# device time: 28420 ns/iter; 1.0668x vs baseline; 1.0668x over previous
import jax
import jax.numpy as jnp
from jax import lax
from jax.experimental import pallas as pl
from jax.experimental.pallas import tpu as pltpu

N_DEV = 32
EPS = 1e-5
N_TOT = 4096 * 128


def kernel(x, Wp):
    xt = jnp.swapaxes(x, 2, 3)
    b, h, c, w = xt.shape
    c2 = Wp.shape[1]

    def body(x_ref, wp_ref, out_ref, gather_buf, send_sems, recv_sems):
        my_pos = lax.axis_index("i")

        xs = x_ref[...]
        s = jnp.sum(xs, axis=(1, 3))
        sq = jnp.sum(xs * xs, axis=(1, 3))
        gather_buf[0] = jnp.concatenate([s, sq], axis=0)

        barrier_sem = pltpu.get_barrier_semaphore()
        for k in range(1, N_DEV):
            peer = lax.rem(my_pos + k, N_DEV)
            pl.semaphore_signal(
                barrier_sem, inc=1,
                device_id=(peer,), device_id_type=pl.DeviceIdType.MESH,
            )
        pl.semaphore_wait(barrier_sem, N_DEV - 1)

        rdmas = []
        for k in range(1, N_DEV):
            peer = lax.rem(my_pos + k, N_DEV)
            rdma = pltpu.make_async_remote_copy(
                src_ref=gather_buf.at[0],
                dst_ref=gather_buf.at[k],
                send_sem=send_sems.at[k],
                recv_sem=recv_sems.at[k],
                device_id=(peer,),
                device_id_type=pl.DeviceIdType.MESH,
            )
            rdma.start()
            rdmas.append(rdma)
        for rdma in rdmas:
            rdma.wait_send()
        for rdma in rdmas:
            rdma.wait_recv()

        tot = jnp.sum(gather_buf[...], axis=0)
        mean = tot[0:b] / N_TOT
        var = tot[b : 2 * b] / N_TOT - mean * mean
        rstd = lax.rsqrt(var + EPS)
        hn = ((xs - mean[:, None, :, None]) * rstd[:, None, :, None]).astype(
            jnp.bfloat16
        )
        a = hn * jax.nn.sigmoid(hn)
        a3 = a.reshape(b * h, c, w)
        wp = wp_ref[...].astype(jnp.bfloat16)
        out = lax.dot_general(
            a3, wp,
            dimension_numbers=(((1,), (0,)), ((), ())),
            preferred_element_type=jnp.float32,
        )
        out_ref[...] = out.reshape(b, h, w, c2).astype(jnp.bfloat16)

    return pl.pallas_call(
        body,
        out_shape=jax.ShapeDtypeStruct((b, h, w, c2), jnp.bfloat16),
        in_specs=[
            pl.BlockSpec(memory_space=pltpu.VMEM),
            pl.BlockSpec(memory_space=pltpu.VMEM),
        ],
        out_specs=pl.BlockSpec(memory_space=pltpu.VMEM),
        scratch_shapes=[
            pltpu.VMEM((N_DEV, 2 * b, c), jnp.float32),
            pltpu.SemaphoreType.DMA((N_DEV,)),
            pltpu.SemaphoreType.DMA((N_DEV,)),
        ],
        compiler_params=pltpu.CompilerParams(
            collective_id=0, vmem_limit_bytes=64 * 1024 * 1024
        ),
    )(xt, Wp)


# device time: 28387 ns/iter; 1.0681x vs baseline; 1.0012x over previous
import jax
import jax.numpy as jnp
from jax import lax
from jax.experimental import pallas as pl
from jax.experimental.pallas import tpu as pltpu

N_DEV = 32
EPS = 1e-5
N_TOT = 4096 * 128


def kernel(x, Wp):
    xt = jnp.swapaxes(x, 2, 3)
    b, h, c, w = xt.shape
    c2 = Wp.shape[1]

    def body(x_ref, wp_ref, out_ref, gather_buf, send_sems, recv_sems):
        my_pos = lax.axis_index("i")

        barrier_sem = pltpu.get_barrier_semaphore()
        for k in range(1, N_DEV):
            peer = lax.rem(my_pos + k, N_DEV)
            pl.semaphore_signal(
                barrier_sem, inc=1,
                device_id=(peer,), device_id_type=pl.DeviceIdType.MESH,
            )

        xs = x_ref[...]
        s = jnp.sum(xs, axis=(1, 3))
        sq = jnp.sum(xs * xs, axis=(1, 3))
        gather_buf[0] = jnp.concatenate([s, sq], axis=0)

        pl.semaphore_wait(barrier_sem, N_DEV - 1)

        rdmas = []
        for k in range(1, N_DEV):
            peer = lax.rem(my_pos + k, N_DEV)
            rdma = pltpu.make_async_remote_copy(
                src_ref=gather_buf.at[0],
                dst_ref=gather_buf.at[k],
                send_sem=send_sems.at[k],
                recv_sem=recv_sems.at[k],
                device_id=(peer,),
                device_id_type=pl.DeviceIdType.MESH,
            )
            rdma.start()
            rdmas.append(rdma)
        for rdma in rdmas:
            rdma.wait_send()
        for rdma in rdmas:
            rdma.wait_recv()

        tot = jnp.sum(gather_buf[...], axis=0)
        mean = tot[0:b] / N_TOT
        var = tot[b : 2 * b] / N_TOT - mean * mean
        rstd = lax.rsqrt(var + EPS)
        hn = ((xs - mean[:, None, :, None]) * rstd[:, None, :, None]).astype(
            jnp.bfloat16
        )
        a = hn * jax.nn.sigmoid(hn)
        a3 = a.reshape(b * h, c, w)
        wp = wp_ref[...].astype(jnp.bfloat16)
        out = lax.dot_general(
            a3, wp,
            dimension_numbers=(((1,), (0,)), ((), ())),
            preferred_element_type=jnp.float32,
        )
        out_ref[...] = out.reshape(b, h, w, c2).astype(jnp.bfloat16)

    return pl.pallas_call(
        body,
        out_shape=jax.ShapeDtypeStruct((b, h, w, c2), jnp.bfloat16),
        in_specs=[
            pl.BlockSpec(memory_space=pltpu.VMEM),
            pl.BlockSpec(memory_space=pltpu.VMEM),
        ],
        out_specs=pl.BlockSpec(memory_space=pltpu.VMEM),
        scratch_shapes=[
            pltpu.VMEM((N_DEV, 2 * b, c), jnp.float32),
            pltpu.SemaphoreType.DMA((N_DEV,)),
            pltpu.SemaphoreType.DMA((N_DEV,)),
        ],
        compiler_params=pltpu.CompilerParams(
            collective_id=0, vmem_limit_bytes=64 * 1024 * 1024
        ),
    )(xt, Wp)
